# Initial kernel scaffold; baseline (speedup 1.0000x reference)
import jax
import jax.numpy as jnp
from jax.experimental import pallas as pl


def kernel(node_features, frac_coords, lattices, edge_index, edge2graph, W_e1, b_e1, W_e2, b_e2, W_n1, b_n1, W_n2, b_n2):
    raise NotImplementedError("write your pallas kernel here")


# speedup vs baseline: 4.0097x; 4.0097x over previous
import functools

import jax
import jax.numpy as jnp
from jax import lax
from jax.experimental import pallas as pl
from jax.experimental.pallas import tpu as pltpu
from jax.experimental.pallas import tpu_sc as plsc

N, E, G, H = 10000, 320000, 256, 128
NC, NS = 2, 16
NW = NC * NS
EPW = E // NW
CH = 80
NCHUNK = EPW // CH


def _silu(x):
    return x * (1.0 / (1.0 + jnp.exp(-x)))


def _tc_prologue(nf, w_src, w_dst, lip16, w_lip16, b_e1):
    def body(nf_ref, ws_ref, wd_ref, lip_ref, wl_ref, be1_ref,
             psrc_ref, pdst_ref, lipb_ref):
        nfv = nf_ref[...]
        psrc_ref[...] = jnp.dot(nfv, ws_ref[...], preferred_element_type=jnp.float32)
        pdst_ref[...] = jnp.dot(nfv, wd_ref[...], preferred_element_type=jnp.float32)
        lipb_ref[...] = (jnp.dot(lip_ref[...], wl_ref[...],
                                 preferred_element_type=jnp.float32)
                         + be1_ref[...])

    return pl.pallas_call(
        body,
        out_shape=(
            jax.ShapeDtypeStruct((N, H), jnp.float32),
            jax.ShapeDtypeStruct((N, H), jnp.float32),
            jax.ShapeDtypeStruct((G, H), jnp.float32),
        ),
    )(nf, w_src, w_dst, lip16, w_lip16, b_e1)


def _sc_gather(psrc, pdst, frac, srcix, dstix):
    mesh = plsc.VectorSubcoreMesh(core_axis_name="c", subcore_axis_name="s")

    @functools.partial(
        pl.kernel,
        out_type=(
            jax.ShapeDtypeStruct((E, H), jnp.float32),
            jax.ShapeDtypeStruct((E * 8,), jnp.float32),
        ),
        mesh=mesh,
        compiler_params=pltpu.CompilerParams(needs_layout_passes=False),
        scratch_types=[
            pltpu.VMEM((CH,), jnp.int32),
            pltpu.VMEM((CH,), jnp.int32),
            pltpu.VMEM((CH, H), jnp.float32),
            pltpu.VMEM((CH, H), jnp.float32),
            pltpu.VMEM((N * 3,), jnp.float32),
            pltpu.VMEM((CH * 8,), jnp.float32),
            pltpu.SemaphoreType.DMA,
            pltpu.SemaphoreType.DMA,
        ],
    )
    def gk(psrc_hbm, pdst_hbm, frac_hbm, six_hbm, dix_hbm,
           pe_out, fd_out, isv, idv, rsv, rdv, fracv, fdv, sem1, sem2):
        wid = lax.axis_index("c") * NS + lax.axis_index("s")
        base0 = wid * EPW
        pltpu.sync_copy(frac_hbm, fracv)

        zero16 = jnp.zeros((16,), jnp.float32)
        lane = lax.iota(jnp.int32, 16)

        def zgrp(t, _):
            fdv[pl.ds(t * 16, 16)] = zero16
            return _

        lax.fori_loop(0, CH * 8 // 16, zgrp, 0)

        def chunk(i, _):
            base = base0 + i * CH
            pltpu.sync_copy(six_hbm.at[pl.ds(base, CH)], isv)
            pltpu.sync_copy(dix_hbm.at[pl.ds(base, CH)], idv)
            cp1 = pltpu.async_copy(psrc_hbm.at[isv], rsv, sem1)
            cp2 = pltpu.async_copy(pdst_hbm.at[idv], rdv, sem2)
            cp1.wait()
            cp2.wait()
            for g in range(CH // 16):
                sl = pl.ds(g * 16, 16)
                rs_ix = isv[sl] * 3
                rd_ix = idv[sl] * 3
                rows8 = (lane + g * 16) * 8
                for c in range(3):
                    fs = plsc.load_gather(fracv, [rs_ix + c])
                    fdd = plsc.load_gather(fracv, [rd_ix + c])
                    d = fdd - fs
                    d = jnp.where(d < 0.0, d + 1.0, d)
                    plsc.store_scatter(fdv, [rows8 + c], d)
            def addrow(r, _):
                for c in range(H // 16):
                    sl2 = pl.ds(c * 16, 16)
                    rsv[r, sl2] = rsv[r, sl2] + rdv[r, sl2]
                return _
            lax.fori_loop(0, CH, addrow, 0)
            pltpu.sync_copy(rsv, pe_out.at[pl.ds(base, CH)])
            pltpu.sync_copy(fdv, fd_out.at[pl.ds(base * 8, CH * 8)])
            return _

        lax.fori_loop(0, NCHUNK, chunk, 0)

    pe, fd_flat = gk(psrc, pdst, frac.reshape(-1), srcix, dstix)
    return pe, fd_flat.reshape(E, 8)


def _tc_edge_mlp(pe, fd, e2g3, lipb, w_fd8, w_e2, b_e2):
    BE = 1280
    GRID = E // BE

    def body(pe_ref, fd_ref, e2g_ref, lipb_ref, wfd_ref, we2_ref, be2_ref, ef_ref):
        gids = e2g_ref[0, 0, :].reshape(BE, 1)
        giota = lax.broadcasted_iota(jnp.int32, (BE, G), 1)
        onehot = jnp.where(gids == giota, 1.0, 0.0).astype(jnp.float32)
        x = (pe_ref[...]
             + jnp.dot(onehot, lipb_ref[...], preferred_element_type=jnp.float32)
             + jnp.dot(fd_ref[...], wfd_ref[...], preferred_element_type=jnp.float32))
        h = _silu(x)
        y = jnp.dot(h, we2_ref[...], preferred_element_type=jnp.float32) + be2_ref[...]
        ef_ref[...] = _silu(y)

    return pl.pallas_call(
        body,
        grid=(GRID,),
        in_specs=[
            pl.BlockSpec((BE, H), lambda i: (i, 0)),
            pl.BlockSpec((BE, 8), lambda i: (i, 0)),
            pl.BlockSpec((1, 1, BE), lambda i: (i, 0, 0)),
            pl.BlockSpec((G, H), lambda i: (0, 0)),
            pl.BlockSpec((8, H), lambda i: (0, 0)),
            pl.BlockSpec((H, H), lambda i: (0, 0)),
            pl.BlockSpec((1, H), lambda i: (0, 0)),
        ],
        out_specs=pl.BlockSpec((BE, H), lambda i: (i, 0)),
        out_shape=jax.ShapeDtypeStruct((E, H), jnp.float32),
    )(pe, fd, e2g3, lipb, w_fd8, w_e2, b_e2)


def _sc_scatter(ef, srcix):
    mesh = plsc.VectorSubcoreMesh(core_axis_name="c", subcore_axis_name="s")
    ZR = 80
    NZC = N // ZR

    @functools.partial(
        pl.kernel,
        out_type=(
            jax.ShapeDtypeStruct((NC, N, H), jnp.float32),
            jax.ShapeDtypeStruct((NC * N,), jnp.float32),
        ),
        mesh=mesh,
        compiler_params=pltpu.CompilerParams(needs_layout_passes=False),
        scratch_types=[
            pltpu.VMEM((CH,), jnp.int32),
            pltpu.VMEM((CH, H), jnp.float32),
            pltpu.VMEM((CH,), jnp.float32),
            pltpu.VMEM((ZR, H), jnp.float32),
            pltpu.VMEM((400,), jnp.float32),
            pltpu.VMEM_SHARED((N, H), jnp.float32),
            pltpu.VMEM_SHARED((N,), jnp.float32),
        ],
    )
    def sk(ef_hbm, six_hbm, sums_out, cnt_out,
           ixv, updv, onesv, zv, zc, accs, cnts):
        cid = lax.axis_index("c")
        sid = lax.axis_index("s")
        zero16 = jnp.zeros((16,), jnp.float32)
        one16 = jnp.ones((16,), jnp.float32)

        def zrow(r, _):
            for c in range(H // 16):
                zv[r, pl.ds(c * 16, 16)] = zero16
            return _
        lax.fori_loop(0, ZR, zrow, 0)

        def zc_fill(t, _):
            zc[pl.ds(t * 16, 16)] = zero16
            return _
        lax.fori_loop(0, 400 // 16, zc_fill, 0)

        def ones_fill(t, _):
            onesv[pl.ds(t * 16, 16)] = one16
            return _
        lax.fori_loop(0, CH // 16, ones_fill, 0)

        def zacc(j, carry):
            k = sid + j * NS

            @pl.when(k < NZC)
            def _zc():
                pltpu.sync_copy(zv, accs.at[pl.ds(k * ZR, ZR)])
            return carry
        lax.fori_loop(0, (NZC + NS - 1) // NS, zacc, 0)

        @pl.when(sid == 0)
        def _():
            def zcnt(k, _):
                pltpu.sync_copy(zc, cnts.at[pl.ds(k * 400, 400)])
                return _
            lax.fori_loop(0, N // 400, zcnt, 0)

        plsc.subcore_barrier()

        def chunk(i, _):
            base = (cid * NS + sid) * EPW + i * CH
            pltpu.sync_copy(six_hbm.at[pl.ds(base, CH)], ixv)
            pltpu.sync_copy(ef_hbm.at[pl.ds(base, CH)], updv)
            pltpu.sync_copy(updv, accs.at[ixv], add=True)
            pltpu.sync_copy(onesv, cnts.at[ixv], add=True)
            return _
        lax.fori_loop(0, NCHUNK, chunk, 0)

        plsc.subcore_barrier()

        def wacc(j, carry):
            k = sid + j * NS

            @pl.when(k < NZC)
            def _wc():
                sl = pl.ds(k * ZR, ZR)
                pltpu.sync_copy(accs.at[sl], zv)
                pltpu.sync_copy(zv, sums_out.at[cid, sl])
            return carry
        lax.fori_loop(0, (NZC + NS - 1) // NS, wacc, 0)

        @pl.when(sid == 0)
        def _():
            def wcnt(k, _):
                pltpu.sync_copy(cnts.at[pl.ds(k * 400, 400)], zc)
                pltpu.sync_copy(zc, cnt_out.at[pl.ds(cid * N + k * 400, 400)])
                return _
            lax.fori_loop(0, N // 400, wcnt, 0)

    sums, cnt_flat = sk(ef, srcix)
    return sums, cnt_flat.reshape(NC, N, 1)


def _tc_epilogue(nf, sums, cnt, w_n1a, w_n1b, b_n1, w_n2, b_n2):
    BN = 1000
    GRID = N // BN

    def body(nf_ref, sums_ref, cnt_ref, wa_ref, wb_ref, bn1_ref, wn2_ref, bn2_ref, out_ref):
        s = sums_ref[0].astype(jnp.float32) + sums_ref[1].astype(jnp.float32)
        c = cnt_ref[0, :, 0] + cnt_ref[1, :, 0]
        agg = s / jnp.maximum(c, 1.0).reshape(BN, 1)
        nfv = nf_ref[...]
        h2 = _silu(jnp.dot(nfv, wa_ref[...], preferred_element_type=jnp.float32)
                   + jnp.dot(agg, wb_ref[...], preferred_element_type=jnp.float32)
                   + bn1_ref[...])
        out_ref[...] = nfv + _silu(
            jnp.dot(h2, wn2_ref[...], preferred_element_type=jnp.float32) + bn2_ref[...])

    return pl.pallas_call(
        body,
        grid=(GRID,),
        in_specs=[
            pl.BlockSpec((BN, H), lambda i: (i, 0)),
            pl.BlockSpec((NC, BN, H), lambda i: (0, i, 0)),
            pl.BlockSpec((NC, BN, 1), lambda i: (0, i, 0)),
            pl.BlockSpec((H, H), lambda i: (0, 0)),
            pl.BlockSpec((H, H), lambda i: (0, 0)),
            pl.BlockSpec((1, H), lambda i: (0, 0)),
            pl.BlockSpec((H, H), lambda i: (0, 0)),
            pl.BlockSpec((1, H), lambda i: (0, 0)),
        ],
        out_specs=pl.BlockSpec((BN, H), lambda i: (i, 0)),
        out_shape=jax.ShapeDtypeStruct((N, H), jnp.float32),
    )(nf, sums, cnt, w_n1a, w_n1b, b_n1, w_n2, b_n2)


def kernel(node_features, frac_coords, lattices, edge_index, edge2graph,
           W_e1, b_e1, W_e2, b_e2, W_n1, b_n1, W_n2, b_n2):
    src = edge_index[0]
    dst = edge_index[1]

    lattice_ips = jnp.matmul(lattices, jnp.swapaxes(lattices, -1, -2))
    lip16 = jnp.pad(lattice_ips.reshape(G, 9), ((0, 0), (0, 7)))
    w_src = W_e1[0:H]
    w_dst = W_e1[H:2 * H]
    w_lip16 = jnp.pad(W_e1[2 * H:2 * H + 9], ((0, 7), (0, 0)))
    w_fd8 = jnp.pad(W_e1[2 * H + 9:2 * H + 12], ((0, 5), (0, 0)))
    e2g3 = edge2graph.reshape(E // 1280, 1, 1280)
    b_e1r = b_e1.reshape(1, H)
    b_e2r = b_e2.reshape(1, H)
    b_n1r = b_n1.reshape(1, H)
    b_n2r = b_n2.reshape(1, H)
    w_n1a = W_n1[0:H]
    w_n1b = W_n1[H:2 * H]

    psrc, pdst, lipb = _tc_prologue(node_features, w_src, w_dst, lip16, w_lip16, b_e1r)
    pe, fd = _sc_gather(psrc, pdst, frac_coords, src, dst)
    ef = _tc_edge_mlp(pe, fd, e2g3, lipb, w_fd8, W_e2, b_e2r)
    sums, cnt = _sc_scatter(ef, src)
    return _tc_epilogue(node_features, sums, cnt, w_n1a, w_n1b, b_n1r, W_n2, b_n2r)
